# Initial kernel scaffold; baseline (speedup 1.0000x reference)
import jax
import jax.numpy as jnp
from jax.experimental import pallas as pl


def kernel(top_k, idx, W, b):
    raise NotImplementedError("write your pallas kernel here")


# speedup vs baseline: 7.2669x; 7.2669x over previous
import functools

import jax
import jax.numpy as jnp
from jax import lax
from jax.experimental import pallas as pl
from jax.experimental.pallas import tpu as pltpu
from jax.experimental.pallas import tpu_sc as plsc

B = 256
K = 4096
H, W = 210, 160
HW = H * W
HP, WP = H + 2, W + 4
CPAD = HP * WP

NC, NS = 2, 16
NW = NC * NS
SPW = B // NW

_mesh = plsc.VectorSubcoreMesh(
    core_axis_name="c", subcore_axis_name="s", num_cores=NC, num_subcores=NS
)


@functools.partial(
    pl.kernel,
    out_type=jax.ShapeDtypeStruct((B, CPAD), jnp.float32),
    mesh=_mesh,
    compiler_params=pltpu.CompilerParams(needs_layout_passes=False),
    scratch_types=[
        pltpu.VMEM((K,), jnp.int32),
        pltpu.VMEM((K,), jnp.float32),
        pltpu.VMEM((CPAD,), jnp.float32),
    ],
)
def _scatter_sc(idx_hbm, val_hbm, out_hbm, idx_v, val_v, canvas_v):
    wid = lax.axis_index("s") * NC + lax.axis_index("c")

    @pl.loop(0, SPW)
    def _sample(j):
        s = wid * SPW + j
        pltpu.sync_copy(idx_hbm.at[s], idx_v)
        pltpu.sync_copy(val_hbm.at[s], val_v)

        zeros = jnp.zeros((16,), jnp.float32)

        @pl.loop(0, CPAD // 16, unroll=8)
        def _zero(i):
            canvas_v[pl.ds(i * 16, 16)] = zeros

        @pl.loop(0, K // 16, unroll=4)
        def _scat(c):
            iv = idx_v[pl.ds(c * 16, 16)]
            vv = val_v[pl.ds(c * 16, 16)]
            row = jnp.right_shift(iv * 26215, 22)
            pidx = iv + row * 4 + (WP + 1)
            plsc.store_scatter(canvas_v, [pidx], vv)

        pltpu.sync_copy(canvas_v, out_hbm.at[s])


BS = 8


def _conv_body(x_ref, w_ref, b_ref, o_ref):
    x = x_ref[...]
    outs = []
    for o in range(3):
        acc = None
        for dy in range(3):
            for dx in range(3):
                t = w_ref[o, dy, dx] * x[:, dy : dy + H, dx : dx + W]
                acc = t if acc is None else acc + t
        acc = acc + b_ref[o]
        outs.append(1.0 / (1.0 + jnp.exp(-acc)))
    o_ref[...] = jnp.stack(outs, axis=1)


_conv_tc = pl.pallas_call(
    _conv_body,
    grid=(B // BS,),
    in_specs=[
        pl.BlockSpec((BS, HP, WP), lambda i: (i, 0, 0)),
        pl.BlockSpec(memory_space=pltpu.SMEM),
        pl.BlockSpec(memory_space=pltpu.SMEM),
    ],
    out_specs=pl.BlockSpec((BS, 3, H, W), lambda i: (i, 0, 0, 0)),
    out_shape=jax.ShapeDtypeStruct((B, 3, H, W), jnp.float32),
)


def kernel(top_k, idx, W_arr, b):
    idx2 = idx.reshape(B, K)
    val2 = top_k.reshape(B, K)
    canvas = _scatter_sc(idx2, val2)
    x3 = canvas.reshape(B, HP, WP)
    return _conv_tc(x3, W_arr.reshape(3, 3, 3), b)


# speedup vs baseline: 19.0590x; 2.6227x over previous
import functools

import jax
import jax.numpy as jnp
from jax import lax
from jax.experimental import pallas as pl
from jax.experimental.pallas import tpu as pltpu
from jax.experimental.pallas import tpu_sc as plsc

B = 256
K = 4096
H, W = 210, 160
HW = H * W
HP, WP = H + 2, W + 32
CPAD = HP * WP
HB = 30
NBLK = H // HB
HB2 = HB + 2
BW2 = HB2 * WP

NC, NS = 2, 16
NW = NC * NS
SPW = B // NW

_mesh = plsc.VectorSubcoreMesh(
    core_axis_name="c", subcore_axis_name="s", num_cores=NC, num_subcores=NS
)


@functools.partial(
    pl.kernel,
    out_type=jax.ShapeDtypeStruct((NBLK * B, BW2), jnp.float32),
    mesh=_mesh,
    compiler_params=pltpu.CompilerParams(needs_layout_passes=False),
    scratch_types=[
        pltpu.VMEM((K,), jnp.int32),
        pltpu.VMEM((K,), jnp.float32),
        pltpu.VMEM((CPAD,), jnp.float32),
    ],
)
def _scatter_sc(idx_hbm, val_hbm, out_hbm, idx_v, val_v, canvas_v):
    wid = lax.axis_index("s") * NC + lax.axis_index("c")

    @pl.loop(0, SPW)
    def _sample(j):
        s = wid * SPW + j
        pltpu.sync_copy(idx_hbm.at[s, 0], idx_v)
        pltpu.sync_copy(val_hbm.at[s, 0], val_v)

        zeros = jnp.zeros((16,), jnp.float32)

        @pl.loop(0, CPAD // 16, unroll=8)
        def _zero(i):
            canvas_v[pl.ds(i * 16, 16)] = zeros

        @pl.loop(0, K // 16, unroll=4)
        def _scat(c):
            iv = idx_v[pl.ds(c * 16, 16)]
            vv = val_v[pl.ds(c * 16, 16)]
            row = jnp.right_shift(iv * 26215, 22)
            pidx = iv + row * (WP - W) + (WP + 1)
            plsc.store_scatter(canvas_v, [pidx], vv)

        for i in range(NBLK):
            pltpu.sync_copy(
                canvas_v.at[pl.ds(i * HB * WP, BW2)], out_hbm.at[i * B + s]
            )


BH = 128


def _conv_body(x_ref, w_ref, b_ref, o_ref):
    x = x_ref[...]
    t = jnp.transpose(x)
    r = t.reshape(HB2, WP, BH)
    outs = []
    for o in range(3):
        acc = None
        for dy in range(3):
            for dx in range(3):
                v = w_ref[o, dy, dx] * r[dy : dy + HB, dx : dx + W, :]
                acc = v if acc is None else acc + v
        acc = acc + b_ref[o]
        outs.append(1.0 / (1.0 + jnp.exp(-acc)))
    o_ref[...] = jnp.stack(outs, axis=0)


_conv_tc = pl.pallas_call(
    _conv_body,
    grid=(NBLK, B // BH),
    in_specs=[
        pl.BlockSpec((BH, BW2), lambda i, h: (i * (B // BH) + h, 0)),
        pl.BlockSpec(memory_space=pltpu.SMEM),
        pl.BlockSpec(memory_space=pltpu.SMEM),
    ],
    out_specs=pl.BlockSpec((3, HB, W, BH), lambda i, h: (0, i, 0, h)),
    out_shape=jax.ShapeDtypeStruct((3, H, W, B), jnp.float32),
)


def kernel(top_k, idx, W_arr, b):
    blocks = _scatter_sc(idx, top_k)
    y = _conv_tc(blocks, W_arr.reshape(3, 3, 3), b)
    return jnp.transpose(y, (3, 0, 1, 2))


# speedup vs baseline: 25.3923x; 1.3323x over previous
import functools

import jax
import jax.numpy as jnp
from jax import lax
from jax.experimental import pallas as pl
from jax.experimental.pallas import tpu as pltpu
from jax.experimental.pallas import tpu_sc as plsc

B = 256
K = 4096
H, W = 210, 160
HW = H * W
HP, WP = H + 2, W + 32
CPAD = HP * WP
HB = 30
NBLK = H // HB
HB2 = HB + 2
BW2 = HB2 * WP

NC, NS = 2, 16
NW = NC * NS
SPW = B // NW

_mesh = plsc.VectorSubcoreMesh(
    core_axis_name="c", subcore_axis_name="s", num_cores=NC, num_subcores=NS
)


@functools.partial(
    pl.kernel,
    out_type=jax.ShapeDtypeStruct((NBLK * B, BW2), jnp.float32),
    mesh=_mesh,
    compiler_params=pltpu.CompilerParams(needs_layout_passes=False),
    scratch_types=[
        pltpu.VMEM((K,), jnp.int32),
        pltpu.VMEM((K,), jnp.float32),
        pltpu.VMEM((K,), jnp.int32),
        pltpu.VMEM((CPAD,), jnp.float32),
    ],
)
def _scatter_sc(idx_hbm, val_hbm, out_hbm, idx_v, val_v, pidx_v, canvas_v):
    wid = lax.axis_index("s") * NC + lax.axis_index("c")

    zeros = jnp.zeros((16,), jnp.float32)

    @pl.loop(0, CPAD // 16, unroll=8)
    def _zero(i):
        canvas_v[pl.ds(i * 16, 16)] = zeros

    @pl.loop(0, SPW)
    def _sample(j):
        s = wid * SPW + j
        pltpu.sync_copy(idx_hbm.at[s, 0], idx_v)
        pltpu.sync_copy(val_hbm.at[s, 0], val_v)

        @pl.loop(0, K // 16, unroll=4)
        def _scat(c):
            iv = idx_v[pl.ds(c * 16, 16)]
            vv = val_v[pl.ds(c * 16, 16)]
            row = jnp.right_shift(iv * 26215, 22)
            pidx = iv + row * (WP - W) + (WP + 1)
            pidx_v[pl.ds(c * 16, 16)] = pidx
            plsc.store_scatter(canvas_v, [pidx], vv)

        for i in range(NBLK):
            pltpu.sync_copy(
                canvas_v.at[pl.ds(i * HB * WP, BW2)], out_hbm.at[i * B + s]
            )

        @pl.loop(0, K // 16, unroll=4)
        def _rezero(c):
            pv = pidx_v[pl.ds(c * 16, 16)]
            plsc.store_scatter(canvas_v, [pv], zeros)


BH = 128


def _conv_body(x_ref, w_ref, b_ref, o_ref, scr_ref):
    x = x_ref[...]
    t = jnp.transpose(x)
    r = t.reshape(HB2, WP, BH)
    for dx in range(3):
        scr_ref[dx] = r[:, dx : dx + W, :]
    outs = []
    for o in range(3):
        acc = None
        for dy in range(3):
            for dx in range(3):
                v = w_ref[o, dy, dx] * scr_ref[dx, dy : dy + HB]
                acc = v if acc is None else acc + v
        acc = acc + b_ref[o]
        outs.append(1.0 / (1.0 + jnp.exp(-acc)))
    o_ref[...] = jnp.stack(outs, axis=0)


_conv_tc = pl.pallas_call(
    _conv_body,
    grid=(NBLK, B // BH),
    in_specs=[
        pl.BlockSpec((BH, BW2), lambda i, h: (i * (B // BH) + h, 0)),
        pl.BlockSpec(memory_space=pltpu.SMEM),
        pl.BlockSpec(memory_space=pltpu.SMEM),
    ],
    out_specs=pl.BlockSpec((3, HB, W, BH), lambda i, h: (0, i, 0, h)),
    out_shape=jax.ShapeDtypeStruct((3, H, W, B), jnp.float32),
    scratch_shapes=[pltpu.VMEM((3, HB2, W, BH), jnp.float32)],
)


def kernel(top_k, idx, W_arr, b):
    blocks = _scatter_sc(idx, top_k)
    y = _conv_tc(blocks, W_arr.reshape(3, 3, 3), b)
    return jnp.transpose(y, (3, 0, 1, 2))


# speedup vs baseline: 29.6801x; 1.1689x over previous
import functools

import jax
import jax.numpy as jnp
from jax import lax
from jax.experimental import pallas as pl
from jax.experimental.pallas import tpu as pltpu
from jax.experimental.pallas import tpu_sc as plsc

B = 256
K = 4096
H, W = 210, 160
HW = H * W
HP, WP = H + 2, W + 32
CPAD = HP * WP
HB = 30
NBLK = H // HB
HB2 = HB + 2
BW2 = HB2 * WP

NC, NS = 2, 16
NW = NC * NS
SPW = B // NW

_mesh = plsc.VectorSubcoreMesh(
    core_axis_name="c", subcore_axis_name="s", num_cores=NC, num_subcores=NS
)

BHALF = B // 2
SPWH = BHALF // NW


def _make_scatter(base):
    @functools.partial(
        pl.kernel,
        out_type=jax.ShapeDtypeStruct((NBLK * BHALF, BW2), jnp.float32),
        mesh=_mesh,
        compiler_params=pltpu.CompilerParams(needs_layout_passes=False),
        scratch_types=[
            pltpu.VMEM((K,), jnp.int32),
            pltpu.VMEM((K,), jnp.float32),
            pltpu.VMEM((K,), jnp.int32),
            pltpu.VMEM((CPAD,), jnp.float32),
        ],
    )
    def _scatter_sc(idx_hbm, val_hbm, out_hbm, idx_v, val_v, pidx_v, canvas_v):
        wid = lax.axis_index("s") * NC + lax.axis_index("c")

        zeros = jnp.zeros((16,), jnp.float32)

        @pl.loop(0, CPAD // 16, unroll=8)
        def _zero(i):
            canvas_v[pl.ds(i * 16, 16)] = zeros

        @pl.loop(0, SPWH)
        def _sample(j):
            sl = wid * SPWH + j
            pltpu.sync_copy(idx_hbm.at[base + sl, 0], idx_v)
            pltpu.sync_copy(val_hbm.at[base + sl, 0], val_v)

            @pl.loop(0, K // 16, unroll=4)
            def _scat(c):
                iv = idx_v[pl.ds(c * 16, 16)]
                vv = val_v[pl.ds(c * 16, 16)]
                row = jnp.right_shift(iv * 26215, 22)
                pidx = iv + row * (WP - W) + (WP + 1)
                pidx_v[pl.ds(c * 16, 16)] = pidx
                plsc.store_scatter(canvas_v, [pidx], vv)

            for i in range(NBLK):
                pltpu.sync_copy(
                    canvas_v.at[pl.ds(i * HB * WP, BW2)],
                    out_hbm.at[i * BHALF + sl],
                )

            @pl.loop(0, K // 16, unroll=4)
            def _rezero(c):
                pv = pidx_v[pl.ds(c * 16, 16)]
                plsc.store_scatter(canvas_v, [pv], zeros)

    return _scatter_sc


_sc_h0 = _make_scatter(0)
_sc_h1 = _make_scatter(BHALF)


BH = 128


def _conv_compute(x_ref, w_ref, b_ref, o_ref, scr_ref):
    x = x_ref[...]
    t = jnp.transpose(x)
    r = t.reshape(HB2, WP, BH)
    for dx in range(3):
        scr_ref[dx] = r[:, dx : dx + W, :]
    outs = []
    for o in range(3):
        acc = None
        for dy in range(3):
            for dx in range(3):
                v = w_ref[o, dy, dx] * scr_ref[dx, dy : dy + HB]
                acc = v if acc is None else acc + v
        acc = acc + b_ref[o]
        outs.append(1.0 / (1.0 + jnp.exp(-acc)))
    o_ref[...] = jnp.stack(outs, axis=0)


def _make_conv(h, aliased):
    in_specs = [
        pl.BlockSpec((BH, BW2), lambda i: (i, 0)),
        pl.BlockSpec(memory_space=pltpu.SMEM),
        pl.BlockSpec(memory_space=pltpu.SMEM),
    ]
    if aliased:
        in_specs.append(pl.BlockSpec(memory_space=pl.ANY))

        def body(x_ref, w_ref, b_ref, y_ref, o_ref, scr_ref):
            del y_ref
            _conv_compute(x_ref, w_ref, b_ref, o_ref, scr_ref)

    else:
        body = _conv_compute
    return pl.pallas_call(
        body,
        grid=(NBLK,),
        in_specs=in_specs,
        out_specs=pl.BlockSpec((3, HB, W, BH), lambda i: (0, i, 0, h)),
        out_shape=jax.ShapeDtypeStruct((3, H, W, B), jnp.float32),
        scratch_shapes=[pltpu.VMEM((3, HB2, W, BH), jnp.float32)],
        input_output_aliases={3: 0} if aliased else {},
    )


_conv_h0 = _make_conv(0, aliased=False)
_conv_h1 = _make_conv(1, aliased=True)


def kernel(top_k, idx, W_arr, b):
    w3 = W_arr.reshape(3, 3, 3)
    o1 = _sc_h0(idx, top_k)
    o2 = _sc_h1(idx, top_k)
    y1 = _conv_h0(o1, w3, b)
    y2 = _conv_h1(o2, w3, b, y1)
    return jnp.transpose(y2, (3, 0, 1, 2))


# speedup vs baseline: 30.2835x; 1.0203x over previous
import functools

import jax
import jax.numpy as jnp
from jax import lax
from jax.experimental import pallas as pl
from jax.experimental.pallas import tpu as pltpu
from jax.experimental.pallas import tpu_sc as plsc

B = 256
K = 4096
H, W = 210, 160
HW = H * W
HP, WP = H + 2, W + 32
CPAD = HP * WP
HB = 30
NBLK = H // HB
HB2 = HB + 2
BW2 = HB2 * WP

NC, NS = 2, 16
NW = NC * NS
SPW = B // NW

_mesh = plsc.VectorSubcoreMesh(
    core_axis_name="c", subcore_axis_name="s", num_cores=NC, num_subcores=NS
)

BHALF = B // 2
SPWH = BHALF // NW


def _make_scatter(base):
    @functools.partial(
        pl.kernel,
        out_type=jax.ShapeDtypeStruct((NBLK * BHALF, BW2), jnp.float32),
        mesh=_mesh,
        compiler_params=pltpu.CompilerParams(needs_layout_passes=False),
        scratch_types=[
            pltpu.VMEM((K,), jnp.int32),
            pltpu.VMEM((K,), jnp.float32),
            pltpu.VMEM((K,), jnp.int32),
            pltpu.VMEM((K,), jnp.int32),
            pltpu.VMEM((CPAD,), jnp.float32),
            pltpu.VMEM((CPAD,), jnp.float32),
            pltpu.SemaphoreType.DMA,
            pltpu.SemaphoreType.DMA,
        ],
    )
    def _scatter_sc(
        idx_hbm, val_hbm, out_hbm, idx_v, val_v, pidx0, pidx1, can0, can1, sem0, sem1
    ):
        wid = lax.axis_index("s") * NC + lax.axis_index("c")
        zeros = jnp.zeros((16,), jnp.float32)
        canvases, pidxs, sems = (can0, can1), (pidx0, pidx1), (sem0, sem1)

        @pl.loop(0, CPAD // 16, unroll=8)
        def _zero0(i):
            can0[pl.ds(i * 16, 16)] = zeros

        @pl.loop(0, CPAD // 16, unroll=8)
        def _zero1(i):
            can1[pl.ds(i * 16, 16)] = zeros

        descs = [None, None]
        for j in range(SPWH):
            p = j % 2
            canvas_v, pidx_v, sem = canvases[p], pidxs[p], sems[p]
            sl = wid * SPWH + j
            pltpu.sync_copy(idx_hbm.at[base + sl, 0], idx_v)
            pltpu.sync_copy(val_hbm.at[base + sl, 0], val_v)

            if descs[p] is not None:
                for d in descs[p]:
                    d.wait()

                @pl.loop(0, K // 16, unroll=4)
                def _rezero(c):
                    pv = pidx_v[pl.ds(c * 16, 16)]
                    plsc.store_scatter(canvas_v, [pv], zeros)

            @pl.loop(0, K // 16, unroll=4)
            def _scat(c):
                iv = idx_v[pl.ds(c * 16, 16)]
                vv = val_v[pl.ds(c * 16, 16)]
                row = jnp.right_shift(iv * 26215, 22)
                pidx = iv + row * (WP - W) + (WP + 1)
                pidx_v[pl.ds(c * 16, 16)] = pidx
                plsc.store_scatter(canvas_v, [pidx], vv)

            descs[p] = [
                pltpu.async_copy(
                    canvas_v.at[pl.ds(i * HB * WP, BW2)],
                    out_hbm.at[i * BHALF + sl],
                    sem,
                )
                for i in range(NBLK)
            ]
        for p in range(2):
            if descs[p] is not None:
                for d in descs[p]:
                    d.wait()

    return _scatter_sc


_sc_h0 = _make_scatter(0)
_sc_h1 = _make_scatter(BHALF)


BH = 128


def _conv_compute(x_ref, w_ref, b_ref, o_ref, scr_ref):
    x = x_ref[...]
    t = jnp.transpose(x)
    r = t.reshape(HB2, WP, BH)
    for dx in range(3):
        scr_ref[dx] = r[:, dx : dx + W, :]
    outs = []
    for o in range(3):
        acc = None
        for dy in range(3):
            for dx in range(3):
                v = w_ref[o, dy, dx] * scr_ref[dx, dy : dy + HB]
                acc = v if acc is None else acc + v
        acc = acc + b_ref[o]
        outs.append(1.0 / (1.0 + jnp.exp(-acc)))
    o_ref[...] = jnp.stack(outs, axis=0)


def _make_conv(h, aliased):
    in_specs = [
        pl.BlockSpec((BH, BW2), lambda i: (i, 0)),
        pl.BlockSpec(memory_space=pltpu.SMEM),
        pl.BlockSpec(memory_space=pltpu.SMEM),
    ]
    if aliased:
        in_specs.append(pl.BlockSpec(memory_space=pl.ANY))

        def body(x_ref, w_ref, b_ref, y_ref, o_ref, scr_ref):
            del y_ref
            _conv_compute(x_ref, w_ref, b_ref, o_ref, scr_ref)

    else:
        body = _conv_compute
    return pl.pallas_call(
        body,
        grid=(NBLK,),
        in_specs=in_specs,
        out_specs=pl.BlockSpec((3, HB, W, BH), lambda i: (0, i, 0, h)),
        out_shape=jax.ShapeDtypeStruct((3, H, W, B), jnp.float32),
        scratch_shapes=[pltpu.VMEM((3, HB2, W, BH), jnp.float32)],
        input_output_aliases={3: 0} if aliased else {},
    )


_conv_h0 = _make_conv(0, aliased=False)
_conv_h1 = _make_conv(1, aliased=True)


def kernel(top_k, idx, W_arr, b):
    w3 = W_arr.reshape(3, 3, 3)
    o1 = _sc_h0(idx, top_k)
    o2 = _sc_h1(idx, top_k)
    y1 = _conv_h0(o1, w3, b)
    y2 = _conv_h1(o2, w3, b, y1)
    return jnp.transpose(y2, (3, 0, 1, 2))


# speedup vs baseline: 30.3548x; 1.0024x over previous
import functools

import jax
import jax.numpy as jnp
from jax import lax
from jax.experimental import pallas as pl
from jax.experimental.pallas import tpu as pltpu
from jax.experimental.pallas import tpu_sc as plsc

B = 256
K = 4096
H, W = 210, 160
HW = H * W
HP, WP = H + 2, W + 32
CPAD = HP * WP
HB = 30
NBLK = H // HB
HB2 = HB + 2
BW2 = HB2 * WP

NC, NS = 2, 16
NW = NC * NS
SPW = B // NW

_mesh = plsc.VectorSubcoreMesh(
    core_axis_name="c", subcore_axis_name="s", num_cores=NC, num_subcores=NS
)

BHALF = B // 2
SPWH = BHALF // NW


def _make_scatter(base):
    @functools.partial(
        pl.kernel,
        out_type=jax.ShapeDtypeStruct((NBLK * BHALF, BW2), jnp.float32),
        mesh=_mesh,
        compiler_params=pltpu.CompilerParams(needs_layout_passes=False),
        scratch_types=[
            pltpu.VMEM((K,), jnp.int32),
            pltpu.VMEM((K,), jnp.float32),
            pltpu.VMEM((K,), jnp.int32),
            pltpu.VMEM((K,), jnp.int32),
            pltpu.VMEM((CPAD,), jnp.float32),
            pltpu.VMEM((CPAD,), jnp.float32),
            pltpu.SemaphoreType.DMA,
            pltpu.SemaphoreType.DMA,
        ],
    )
    def _scatter_sc(
        idx_hbm, val_hbm, out_hbm, idx_v, val_v, pidx0, pidx1, can0, can1, sem0, sem1
    ):
        wid = lax.axis_index("s") * NC + lax.axis_index("c")
        zeros = jnp.zeros((16,), jnp.float32)
        canvases, pidxs, sems = (can0, can1), (pidx0, pidx1), (sem0, sem1)

        @pl.loop(0, CPAD // 16, unroll=8)
        def _zero0(i):
            can0[pl.ds(i * 16, 16)] = zeros

        @pl.loop(0, CPAD // 16, unroll=8)
        def _zero1(i):
            can1[pl.ds(i * 16, 16)] = zeros

        descs = [None, None]
        for j in range(SPWH):
            p = j % 2
            canvas_v, pidx_v, sem = canvases[p], pidxs[p], sems[p]
            sl = wid * SPWH + j
            pltpu.sync_copy(idx_hbm.at[base + sl, 0], idx_v)
            pltpu.sync_copy(val_hbm.at[base + sl, 0], val_v)

            if descs[p] is not None:
                for d in descs[p]:
                    d.wait()

                @pl.loop(0, K // 16, unroll=4)
                def _rezero(c):
                    pv = pidx_v[pl.ds(c * 16, 16)]
                    plsc.store_scatter(canvas_v, [pv], zeros)

            @pl.loop(0, K // 16, unroll=4)
            def _scat(c):
                iv = idx_v[pl.ds(c * 16, 16)]
                vv = val_v[pl.ds(c * 16, 16)]
                row = jnp.right_shift(iv * 26215, 22)
                pidx = iv + row * (WP - W) + (WP + 1)
                pidx_v[pl.ds(c * 16, 16)] = pidx
                plsc.store_scatter(canvas_v, [pidx], vv)

            descs[p] = [
                pltpu.async_copy(
                    canvas_v.at[pl.ds(i * HB * WP, BW2)],
                    out_hbm.at[i * BHALF + sl],
                    sem,
                )
                for i in range(NBLK)
            ]
        for p in range(2):
            if descs[p] is not None:
                for d in descs[p]:
                    d.wait()

    return _scatter_sc


_sc_h0 = _make_scatter(0)
_sc_h1 = _make_scatter(BHALF)


BH = 128


def _conv_compute(x_ref, w_ref, b_ref, o_ref, scr_ref):
    x = x_ref[...]
    t = jnp.transpose(x)
    r = t.reshape(HB2, WP, BH)
    for dx in (1, 2):
        scr_ref[dx - 1] = r[:, dx : dx + W, :]
    for o in range(3):
        acc = None
        for dy in range(3):
            for dx in range(3):
                src = r[:, 0:W, :] if dx == 0 else scr_ref[dx - 1]
                v = w_ref[o, dy, dx] * src[dy : dy + HB]
                acc = v if acc is None else acc + v
        acc = acc + b_ref[o]
        o_ref[o] = 1.0 / (1.0 + jnp.exp(-acc))


def _make_conv(h, aliased):
    in_specs = [
        pl.BlockSpec((BH, BW2), lambda i: (i, 0)),
        pl.BlockSpec(memory_space=pltpu.SMEM),
        pl.BlockSpec(memory_space=pltpu.SMEM),
    ]
    if aliased:
        in_specs.append(pl.BlockSpec(memory_space=pl.ANY))

        def body(x_ref, w_ref, b_ref, y_ref, o_ref, scr_ref):
            del y_ref
            _conv_compute(x_ref, w_ref, b_ref, o_ref, scr_ref)

    else:
        body = _conv_compute
    return pl.pallas_call(
        body,
        grid=(NBLK,),
        in_specs=in_specs,
        out_specs=pl.BlockSpec((3, HB, W, BH), lambda i: (0, i, 0, h)),
        out_shape=jax.ShapeDtypeStruct((3, H, W, B), jnp.float32),
        scratch_shapes=[pltpu.VMEM((2, HB2, W, BH), jnp.float32)],
        input_output_aliases={3: 0} if aliased else {},
    )


_conv_h0 = _make_conv(0, aliased=False)
_conv_h1 = _make_conv(1, aliased=True)


def kernel(top_k, idx, W_arr, b):
    w3 = W_arr.reshape(3, 3, 3)
    o1 = _sc_h0(idx, top_k)
    o2 = _sc_h1(idx, top_k)
    y1 = _conv_h0(o1, w3, b)
    y2 = _conv_h1(o2, w3, b, y1)
    return jnp.transpose(y2, (3, 0, 1, 2))
